# Initial kernel scaffold; baseline (speedup 1.0000x reference)
import jax
import jax.numpy as jnp
from jax.experimental import pallas as pl


def kernel(x, W_in, b_in, W1, g1, be1, W2, g2, be2, W3, g3, be3, W4, g4, be4):
    raise NotImplementedError("write your pallas kernel here")


# speedup vs baseline: 1.3885x; 1.3885x over previous
import functools

import jax
import jax.numpy as jnp
from jax.experimental import pallas as pl

DGK = 16
NPOINTS = 2048
DOWN_NUM = 256

def _knn_kernel(q_ref, r_ref, idx_ref, *, k):
    q = q_ref[0]
    r = r_ref[0]
    dot = jnp.dot(q, r.T, preferred_element_type=jnp.float32)
    qq = jnp.sum(q * q, axis=1, keepdims=True)
    rr = jnp.sum(r * r, axis=1, keepdims=True)
    d = qq - 2.0 * dot + rr.T
    nk = d.shape[1]
    col = jax.lax.broadcasted_iota(jnp.int32, d.shape, 1)
    for i in range(k):
        dmin = jnp.min(d, axis=1, keepdims=True)
        cand = jnp.where(d <= dmin, col, nk)
        sel = jnp.min(cand, axis=1).astype(jnp.int32)
        idx_ref[0, :, i] = sel
        d = jnp.where(col == sel[:, None], 1e30, d)


def _knn_pallas(coor_q, coor_k, k=DGK, qblk=256):
    B, _, Nq = coor_q.shape
    Nk = coor_k.shape[2]
    qblk = min(qblk, Nq)
    q = jnp.transpose(coor_q, (0, 2, 1))
    r = jnp.transpose(coor_k, (0, 2, 1))
    qpad = jnp.pad(q, ((0, 0), (0, 0), (0, 5)))
    rpad = jnp.pad(r, ((0, 0), (0, 0), (0, 5)))
    grid = (B, Nq // qblk)
    return pl.pallas_call(
        functools.partial(_knn_kernel, k=k),
        grid=grid,
        in_specs=[
            pl.BlockSpec((1, qblk, 8), lambda b, i: (b, i, 0)),
            pl.BlockSpec((1, Nk, 8), lambda b, i: (b, 0, 0)),
        ],
        out_specs=pl.BlockSpec((1, qblk, k), lambda b, i: (b, i, 0)),
        out_shape=jax.ShapeDtypeStruct((B, Nq, k), jnp.int32),
    )(qpad, rpad)


def _mm_kernel(w_ref, f_ref, o_ref):
    o_ref[0] = jnp.dot(w_ref[...], f_ref[0],
                       preferred_element_type=jnp.float32)


def _mm_pallas(W, feat, mblk=4096):
    B, C2, Nq, K = feat.shape
    Co = W.shape[0]
    M = Nq * K
    mblk = min(mblk, M)
    f2 = feat.reshape(B, C2, M)
    grid = (B, M // mblk)
    out = pl.pallas_call(
        _mm_kernel,
        grid=grid,
        in_specs=[
            pl.BlockSpec((Co, C2), lambda b, i: (0, 0)),
            pl.BlockSpec((1, C2, mblk), lambda b, i: (b, 0, i)),
        ],
        out_specs=pl.BlockSpec((1, Co, mblk), lambda b, i: (b, 0, i)),
        out_shape=jax.ShapeDtypeStruct((B, Co, M), jnp.float32),
    )(W, f2)
    return out.reshape(B, Co, Nq, K)


def _get_graph_feature(coor_q, x_q, coor_k, x_k):
    idx = _knn_pallas(coor_q, coor_k)
    xk_t = x_k.transpose(0, 2, 1)
    feat = jax.vmap(lambda xk, ii: xk[ii])(xk_t, idx)
    feat = feat.transpose(0, 3, 1, 2)
    xq = x_q[:, :, :, None]
    return jnp.concatenate([feat - xq, jnp.broadcast_to(xq, feat.shape)],
                           axis=1)


def _group_norm(x, gamma, beta, groups=4, eps=1e-5):
    B, C, H, W = x.shape
    xg = x.reshape(B, groups, C // groups, H, W)
    m = jnp.mean(xg, axis=(2, 3, 4), keepdims=True)
    v = jnp.var(xg, axis=(2, 3, 4), keepdims=True)
    xg = (xg - m) / jnp.sqrt(v + eps)
    x = xg.reshape(B, C, H, W)
    return x * gamma[None, :, None, None] + beta[None, :, None, None]


def _conv_block(f, W, g, b):
    f = _mm_pallas(W, f)
    f = _group_norm(f, g, b)
    return jnp.where(f >= 0, f, 0.2 * f)


def _fps(xyz, n_samples):
    B, N, _ = xyz.shape
    dists0 = jnp.full((B, N), 1e10, dtype=xyz.dtype)
    init_last = jnp.zeros((B,), jnp.int32)

    def step(carry, _):
        dists, last = carry
        last_pt = xyz[jnp.arange(B), last]
        d = jnp.sum((xyz - last_pt[:, None, :]) ** 2, axis=-1)
        dists = jnp.minimum(dists, d)
        nxt = jnp.argmax(dists, axis=-1).astype(jnp.int32)
        return (dists, nxt), nxt

    _, rest = jax.lax.scan(step, (dists0, init_last), None,
                           length=n_samples - 1)
    return jnp.concatenate([init_last[:, None], rest.T], axis=1)


def _fps_downsample(coor, x, num_group):
    xyz = coor.transpose(0, 2, 1)
    fi = _fps(xyz, num_group)
    combined = jnp.concatenate([coor, x], axis=1)
    newc = jax.vmap(lambda c, ii: c[:, ii])(combined, fi)
    return newc[:, :3], newc[:, 3:]


def kernel(x, W_in, b_in, W1, g1, be1, W2, g2, be2, W3, g3, be3, W4, g4, be4):
    coor = x
    f = jnp.einsum('oi,bin->bon', W_in, x) + b_in[None, :, None]
    inpc_f = f
    f = _get_graph_feature(coor, f, coor, f)
    f = _conv_block(f, W1, g1, be1)
    f = jnp.max(f, axis=-1)
    coor_q, f_q = _fps_downsample(coor, f, NPOINTS // 2)
    f = _get_graph_feature(coor_q, f_q, coor, f)
    f = _conv_block(f, W2, g2, be2)
    f = jnp.max(f, axis=-1)
    xyz1, point1 = coor_q, f
    coor = coor_q
    f = _get_graph_feature(coor, f, coor, f)
    f = _conv_block(f, W3, g3, be3)
    f = jnp.max(f, axis=-1)
    coor_q, f_q = _fps_downsample(coor, f, DOWN_NUM)
    f = _get_graph_feature(coor_q, f_q, coor, f)
    f = _conv_block(f, W4, g4, be4)
    f = jnp.max(f, axis=-1)
    return (coor_q, f, xyz1, point1, inpc_f)
